# baseline (speedup 1.0000x reference)
import jax
import jax.numpy as jnp
from jax.experimental import pallas as pl


def kernel(inputs, weight):
    *batches, d1, d2 = inputs.shape
    nbatch = 1
    for b in batches:
        nbatch *= b

    block_rows = 1024
    nblocks = d1 // block_rows

    def body(w_ref, o_ref):
        o_ref[...] = w_ref[...][None]

    out = pl.pallas_call(
        body,
        grid=(nblocks, nbatch),
        in_specs=[pl.BlockSpec((block_rows, d2), lambda i, j: (i, 0))],
        out_specs=pl.BlockSpec((1, block_rows, d2), lambda i, j: (j, i, 0)),
        out_shape=jax.ShapeDtypeStruct((nbatch, d1, d2), weight.dtype),
    )(weight)

    return out.reshape(tuple(batches) + (d1, d2))


# speedup vs baseline: 1.2272x; 1.2272x over previous
import jax
import jax.numpy as jnp
from jax.experimental import pallas as pl


def kernel(inputs, weight):
    *batches, d1, d2 = inputs.shape
    nbatch = 1
    for b in batches:
        nbatch *= b

    block_rows = 1024
    while d1 % block_rows:
        block_rows //= 2
    nblocks = d1 // block_rows

    def body(w_ref, o_ref):
        o_ref[...] = jnp.broadcast_to(w_ref[...][None], (nbatch, block_rows, d2))

    out = pl.pallas_call(
        body,
        grid=(nblocks,),
        in_specs=[pl.BlockSpec((block_rows, d2), lambda i: (i, 0))],
        out_specs=pl.BlockSpec((nbatch, block_rows, d2), lambda i: (0, i, 0)),
        out_shape=jax.ShapeDtypeStruct((nbatch, d1, d2), weight.dtype),
    )(weight)

    return out.reshape(tuple(batches) + (d1, d2))
